# Initial kernel scaffold; baseline (speedup 1.0000x reference)
import jax
import jax.numpy as jnp
from jax.experimental import pallas as pl


def kernel(input, edge_index, adj_vals, W1, W2, W3, Dw1, Db1, Dw2, Db2, Dw3, Db3):
    raise NotImplementedError("write your pallas kernel here")


# speedup vs baseline: 6.9756x; 6.9756x over previous
import functools

import jax
import jax.numpy as jnp
from jax import lax
from jax.experimental import pallas as pl
from jax.experimental.pallas import tpu as pltpu
from jax.experimental.pallas import tpu_sc as plsc

N = 10000
E = 320000
D_IN = 128
H1 = 64
H2 = 32

NC = 2
NS = 16
NW = NC * NS
EPW = E // NW
EK = 80
NCHUNK = EPW // EK
ROWS_PW = 624
ROWS_TAIL = N - NS * ROWS_PW

_sc_mesh = plsc.VectorSubcoreMesh(core_axis_name="c", subcore_axis_name="s")


def _spmm_sc_body(sup_hbm, src_hbm, dst_hbm, val_hbm, zeros_hbm, out_hbm,
                  src_v, dst_v, val_v, rows_v, acc_sh, sem):
    c = lax.axis_index("c")
    s = lax.axis_index("s")
    wid = c * NS + s
    pltpu.sync_copy(zeros_hbm.at[pl.ds(s * ROWS_PW, ROWS_PW)],
                    acc_sh.at[pl.ds(s * ROWS_PW, ROWS_PW)])

    @pl.when(s == 0)
    def _zero_tail():
        pltpu.sync_copy(zeros_hbm.at[pl.ds(NS * ROWS_PW, ROWS_TAIL)],
                        acc_sh.at[pl.ds(NS * ROWS_PW, ROWS_TAIL)])
    pltpu.sync_copy(src_hbm.at[wid], src_v)
    pltpu.sync_copy(dst_hbm.at[wid], dst_v)
    pltpu.sync_copy(val_hbm.at[wid], val_v)
    plsc.subcore_barrier()

    @pl.loop(0, NCHUNK)
    def _chunk(j):
        pltpu.async_copy(sup_hbm.at[src_v.at[j]], rows_v, sem).wait()

        for g in range(EK // 16):
            vals16 = val_v[j, pl.ds(g * 16, 16)]
            for u in range(16):
                v = vals16[u]
                row = g * 16 + u
                for f in range(H1 // 16):
                    sl = (row, pl.ds(f * 16, 16))
                    rows_v[sl] = rows_v[sl] * v

        pltpu.sync_copy(rows_v, acc_sh.at[dst_v.at[j]], add=True)

    plsc.subcore_barrier()
    pltpu.sync_copy(acc_sh.at[pl.ds(s * ROWS_PW, ROWS_PW)],
                    out_hbm.at[c, pl.ds(s * ROWS_PW, ROWS_PW)])

    @pl.when(s == 0)
    def _write_tail():
        pltpu.sync_copy(acc_sh.at[pl.ds(NS * ROWS_PW, ROWS_TAIL)],
                        out_hbm.at[c, pl.ds(NS * ROWS_PW, ROWS_TAIL)])


@jax.jit
def _spmm_sc(sup, src2d, dst2d, val2d, zeros):
    kfn = pl.kernel(
        _spmm_sc_body,
        out_type=jax.ShapeDtypeStruct((NC, N, H1), jnp.float32),
        mesh=_sc_mesh,
        scratch_types=[
            pltpu.VMEM((NCHUNK, EK), jnp.int32),
            pltpu.VMEM((NCHUNK, EK), jnp.int32),
            pltpu.VMEM((NCHUNK, EK), jnp.float32),
            pltpu.VMEM((EK, H1), jnp.float32),
            pltpu.VMEM_SHARED((N, H1), jnp.float32),
            pltpu.SemaphoreType.DMA,
        ],
        compiler_params=pltpu.CompilerParams(use_tc_tiling_on_sc=False),
    )
    return kfn(sup, src2d, dst2d, val2d, zeros)


def _mm_body(x_ref, w_ref, o_ref):
    o_ref[...] = jnp.dot(x_ref[...], w_ref[...],
                         preferred_element_type=jnp.float32,
                         precision=lax.Precision.HIGHEST)


@jax.jit
def _tc_xw1(x, w1):
    return pl.pallas_call(
        _mm_body,
        out_shape=jax.ShapeDtypeStruct((N, H1), jnp.float32),
    )(x, w1)


def _relu_sum_body(p_ref, o_ref):
    o_ref[...] = jnp.maximum(p_ref[0] + p_ref[1], 0.0)


@jax.jit
def _tc_relu_sum(p):
    return pl.pallas_call(
        _relu_sum_body,
        out_shape=jax.ShapeDtypeStruct((N, H1), jnp.float32),
    )(p)


def _epilogue_body(p_ref, w2_ref, w3_ref, dw1_ref, db1_ref, dw2_ref, db2_ref,
                   dw3_ref, db3_ref, zr_ref, mu_ref, lv_ref, dr_ref, df_ref):
    s = p_ref[0] + p_ref[1]
    dot = functools.partial(jnp.dot, preferred_element_type=jnp.float32,
                            precision=lax.Precision.HIGHEST)
    mu = dot(s, w2_ref[...])
    mu_ref[...] = mu
    lv_ref[...] = dot(s, w3_ref[...])

    def disc(z):
        h = jnp.maximum(dot(z, dw1_ref[...]) + db1_ref[...], 0.0)
        h = jnp.maximum(dot(h, dw2_ref[...]) + db2_ref[...], 0.0)
        return dot(h, dw3_ref[...]) + db3_ref[...]

    dr_ref[...] = disc(zr_ref[...])
    df_ref[...] = disc(mu)


RB = 1000


@jax.jit
def _tc_epilogue(p, w2, w3, dw1, db1, dw2, db2, dw3, db3, z_real):
    full = lambda shape: pl.BlockSpec(shape, lambda i: tuple(0 for _ in shape))
    return pl.pallas_call(
        _epilogue_body,
        grid=(N // RB,),
        in_specs=[
            pl.BlockSpec((2, RB, H1), lambda i: (0, i, 0)),
            full((H1, H2)), full((H1, H2)),
            full((H2, H1)), full((1, H1)),
            full((H1, H2)), full((1, H2)),
            full((H2, 1)), full((1, 1)),
            pl.BlockSpec((RB, H2), lambda i: (i, 0)),
        ],
        out_specs=(
            pl.BlockSpec((RB, H2), lambda i: (i, 0)),
            pl.BlockSpec((RB, H2), lambda i: (i, 0)),
            pl.BlockSpec((RB, 1), lambda i: (i, 0)),
            pl.BlockSpec((RB, 1), lambda i: (i, 0)),
        ),
        out_shape=(
            jax.ShapeDtypeStruct((N, H2), jnp.float32),
            jax.ShapeDtypeStruct((N, H2), jnp.float32),
            jax.ShapeDtypeStruct((N, 1), jnp.float32),
            jax.ShapeDtypeStruct((N, 1), jnp.float32),
        ),
    )(p, w2, w3, dw1, db1.reshape(1, H1), dw2, db2.reshape(1, H2),
      dw3, db3.reshape(1, 1), z_real)


BM = 200


def _decoder_body(a_ref, b_ref, o_ref):
    a = a_ref[...].astype(jnp.bfloat16)
    b = b_ref[...].astype(jnp.bfloat16)
    o_ref[...] = lax.dot_general(a, b, (((1,), (1,)), ((), ())),
                                 preferred_element_type=jnp.float32)


@jax.jit
def _tc_decoder(mu):
    return pl.pallas_call(
        _decoder_body,
        grid=(N // BM,),
        in_specs=[
            pl.BlockSpec((BM, H2), lambda i: (i, 0)),
            pl.BlockSpec((N, H2), lambda i: (0, 0)),
        ],
        out_specs=pl.BlockSpec((BM, N), lambda i: (i, 0)),
        out_shape=jax.ShapeDtypeStruct((N, N), jnp.float32),
    )(mu, mu)


def kernel(input, edge_index, adj_vals, W1, W2, W3,
           Dw1, Db1, Dw2, Db2, Dw3, Db3):
    src2d = edge_index[0].reshape(NW, NCHUNK, EK)
    dst2d = edge_index[1].reshape(NW, NCHUNK, EK)
    val2d = adj_vals.reshape(NW, NCHUNK, EK)
    zeros = jnp.zeros((N, H1), jnp.float32)
    z_real = jax.random.normal(jax.random.key(1), (N, H2), dtype=jnp.float32)

    xw1 = _tc_xw1(input, W1)
    p1 = _spmm_sc(xw1, src2d, dst2d, val2d, zeros)
    h1 = _tc_relu_sum(p1)
    p2 = _spmm_sc(h1, src2d, dst2d, val2d, zeros)
    mu, logvar, dis_real, dis_fake = _tc_epilogue(
        p2, W2, W3, Dw1, Db1, Dw2, Db2, Dw3, Db3, z_real)
    adj_rec = _tc_decoder(mu)
    return (adj_rec, dis_real, dis_fake, mu, logvar)


# speedup vs baseline: 10.8465x; 1.5549x over previous
import functools

import jax
import jax.numpy as jnp
from jax import lax
from jax.experimental import pallas as pl
from jax.experimental.pallas import tpu as pltpu
from jax.experimental.pallas import tpu_sc as plsc

N = 10000
E = 320000
D_IN = 128
H1 = 64
H2 = 32

NC = 2
NS = 16
NW = NC * NS
EPW = E // NW
EK = 80
NCHUNK = EPW // EK
ROWS_PW = 624
ROWS_TAIL = N - NS * ROWS_PW

_sc_mesh = plsc.VectorSubcoreMesh(core_axis_name="c", subcore_axis_name="s")


def _scale_rows(rows_ref, val_ref, chunk):
    for g in range(EK // 16):
        vals16 = val_ref[chunk, pl.ds(g * 16, 16)]
        for u in range(16):
            v = vals16[u]
            row = g * 16 + u
            for f in range(H1 // 16):
                sl = (row, pl.ds(f * 16, 16))
                rows_ref[sl] = rows_ref[sl] * v


def _spmm_body(sup_hbm, src_hbm, dst_hbm, val_hbm, zeros_hbm, out_hbm,
               src_v, dst_v, val_v, rows0, rows1, acc_sh, gsem0, gsem1):
    c = lax.axis_index("c")
    s = lax.axis_index("s")
    wid = c * NS + s
    pltpu.sync_copy(zeros_hbm.at[pl.ds(s * ROWS_PW, ROWS_PW)],
                    acc_sh.at[pl.ds(s * ROWS_PW, ROWS_PW)])

    @pl.when(s == 0)
    def _zero_tail():
        pltpu.sync_copy(zeros_hbm.at[pl.ds(NS * ROWS_PW, ROWS_TAIL)],
                        acc_sh.at[pl.ds(NS * ROWS_PW, ROWS_TAIL)])

    pltpu.sync_copy(src_hbm.at[wid], src_v)
    pltpu.sync_copy(dst_hbm.at[wid], dst_v)
    pltpu.sync_copy(val_hbm.at[wid], val_v)
    plsc.subcore_barrier()

    rows = (rows0, rows1)
    gsems = (gsem0, gsem1)

    def _process(chunk, buf, gsem):
        pltpu.make_async_copy(sup_hbm.at[src_v.at[chunk]], buf, gsem).wait()
        _scale_rows(buf, val_v, chunk)
        pltpu.sync_copy(buf, acc_sh.at[dst_v.at[chunk]], add=True)

    pltpu.async_copy(sup_hbm.at[src_v.at[0]], rows0, gsem0)
    pltpu.async_copy(sup_hbm.at[src_v.at[1]], rows1, gsem1)

    @pl.loop(0, NCHUNK - 1, step=2)
    def _pair(j):
        for b in range(2):
            chunk = j + b
            _process(chunk, rows[b], gsems[b])

            @pl.when(chunk + 2 < NCHUNK)
            def _next():
                pltpu.async_copy(sup_hbm.at[src_v.at[chunk + 2]], rows[b],
                                 gsems[b])

    _process(NCHUNK - 1, rows[(NCHUNK - 1) % 2], gsems[(NCHUNK - 1) % 2])

    plsc.subcore_barrier()
    pltpu.sync_copy(acc_sh.at[pl.ds(s * ROWS_PW, ROWS_PW)],
                    out_hbm.at[c, pl.ds(s * ROWS_PW, ROWS_PW)])

    @pl.when(s == 0)
    def _write_tail():
        pltpu.sync_copy(acc_sh.at[pl.ds(NS * ROWS_PW, ROWS_TAIL)],
                        out_hbm.at[c, pl.ds(NS * ROWS_PW, ROWS_TAIL)])


@jax.jit
def _spmm_sc(sup, src3d, dst3d, val3d, zeros):
    kfn = pl.kernel(
        _spmm_body,
        out_type=jax.ShapeDtypeStruct((NC, N, H1), jnp.float32),
        mesh=_sc_mesh,
        scratch_types=[
            pltpu.VMEM((NCHUNK, EK), jnp.int32),
            pltpu.VMEM((NCHUNK, EK), jnp.int32),
            pltpu.VMEM((NCHUNK, EK), jnp.float32),
            pltpu.VMEM((EK, H1), jnp.float32),
            pltpu.VMEM((EK, H1), jnp.float32),
            pltpu.VMEM_SHARED((N, H1), jnp.float32),
            pltpu.SemaphoreType.DMA,
            pltpu.SemaphoreType.DMA,
        ],
        compiler_params=pltpu.CompilerParams(use_tc_tiling_on_sc=False),
    )
    return kfn(sup, src3d, dst3d, val3d, zeros)


def _mm_body(x_ref, w_ref, o_ref):
    o_ref[...] = jnp.dot(x_ref[...], w_ref[...],
                         preferred_element_type=jnp.float32,
                         precision=lax.Precision.DEFAULT)


@jax.jit
def _tc_xw1(x, w1):
    return pl.pallas_call(
        _mm_body,
        out_shape=jax.ShapeDtypeStruct((N, H1), jnp.float32),
    )(x, w1)


def _relu_sum_body(p_ref, o_ref):
    o_ref[...] = jnp.maximum(p_ref[0] + p_ref[1], 0.0)


@jax.jit
def _tc_relu_sum(p):
    return pl.pallas_call(
        _relu_sum_body,
        out_shape=jax.ShapeDtypeStruct((N, H1), jnp.float32),
    )(p)


def _epilogue_body(p_ref, w2_ref, w3_ref, dw1_ref, db1_ref, dw2_ref, db2_ref,
                   dw3_ref, db3_ref, zr_ref, mu_ref, lv_ref, dr_ref, df_ref):
    s = p_ref[0] + p_ref[1]
    dot3 = functools.partial(jnp.dot, preferred_element_type=jnp.float32,
                             precision=lax.Precision.DEFAULT)
    mu = dot3(s, w2_ref[...])
    mu_ref[...] = mu
    lv_ref[...] = dot3(s, w3_ref[...])

    def bdot(a, b):
        return jnp.dot(a.astype(jnp.bfloat16), b.astype(jnp.bfloat16),
                       preferred_element_type=jnp.float32)

    def disc(z):
        h = jnp.maximum(bdot(z, dw1_ref[...]) + db1_ref[...], 0.0)
        h = jnp.maximum(bdot(h, dw2_ref[...]) + db2_ref[...], 0.0)
        return bdot(h, dw3_ref[...]) + db3_ref[...]

    dr_ref[...] = disc(zr_ref[...])
    df_ref[...] = disc(mu)


RB = 2000


@jax.jit
def _tc_epilogue(p, w2, w3, dw1, db1, dw2, db2, dw3, db3, z_real):
    full = lambda shape: pl.BlockSpec(shape, lambda i: tuple(0 for _ in shape))
    return pl.pallas_call(
        _epilogue_body,
        grid=(N // RB,),
        in_specs=[
            pl.BlockSpec((2, RB, H1), lambda i: (0, i, 0)),
            full((H1, H2)), full((H1, H2)),
            full((H2, H1)), full((1, H1)),
            full((H1, H2)), full((1, H2)),
            full((H2, 1)), full((1, 1)),
            pl.BlockSpec((RB, H2), lambda i: (i, 0)),
        ],
        out_specs=(
            pl.BlockSpec((RB, H2), lambda i: (i, 0)),
            pl.BlockSpec((RB, H2), lambda i: (i, 0)),
            pl.BlockSpec((RB, 1), lambda i: (i, 0)),
            pl.BlockSpec((RB, 1), lambda i: (i, 0)),
        ),
        out_shape=(
            jax.ShapeDtypeStruct((N, H2), jnp.float32),
            jax.ShapeDtypeStruct((N, H2), jnp.float32),
            jax.ShapeDtypeStruct((N, 1), jnp.float32),
            jax.ShapeDtypeStruct((N, 1), jnp.float32),
        ),
    )(p, w2, w3, dw1, db1.reshape(1, H1), dw2, db2.reshape(1, H2),
      dw3, db3.reshape(1, 1), z_real)


BM = 200


def _decoder_body(a_ref, b_ref, o_ref):
    a = a_ref[...].astype(jnp.bfloat16)
    b = b_ref[...].astype(jnp.bfloat16)
    o_ref[...] = lax.dot_general(a, b, (((1,), (1,)), ((), ())),
                                 preferred_element_type=jnp.float32)


@jax.jit
def _tc_decoder(mu):
    return pl.pallas_call(
        _decoder_body,
        grid=(N // BM,),
        in_specs=[
            pl.BlockSpec((BM, H2), lambda i: (i, 0)),
            pl.BlockSpec((N, H2), lambda i: (0, 0)),
        ],
        out_specs=pl.BlockSpec((BM, N), lambda i: (i, 0)),
        out_shape=jax.ShapeDtypeStruct((N, N), jnp.float32),
    )(mu, mu)


def kernel(input, edge_index, adj_vals, W1, W2, W3,
           Dw1, Db1, Dw2, Db2, Dw3, Db3):
    src3d = edge_index[0].reshape(NW, NCHUNK, EK)
    dst3d = edge_index[1].reshape(NW, NCHUNK, EK)
    val3d = adj_vals.reshape(NW, NCHUNK, EK)
    zeros = jnp.zeros((N, H1), jnp.float32)
    z_real = jax.random.normal(jax.random.key(1), (N, H2), dtype=jnp.float32)

    xw1 = _tc_xw1(input, W1)
    p1 = _spmm_sc(xw1, src3d, dst3d, val3d, zeros)
    h1 = _tc_relu_sum(p1)
    p2 = _spmm_sc(h1, src3d, dst3d, val3d, zeros)
    mu, logvar, dis_real, dis_fake = _tc_epilogue(
        p2, W2, W3, Dw1, Db1, Dw2, Db2, Dw3, Db3, z_real)
    adj_rec = _tc_decoder(mu)
    return (adj_rec, dis_real, dis_fake, mu, logvar)
